# baseline (speedup 1.0000x reference)
import jax
import jax.numpy as jnp
from jax.experimental import pallas as pl
from jax.experimental.pallas import tpu as pltpu

B, K, L, H = 64, 64, 64, 64


def _fused_kernel(h_ue_ref, h_ap_ref, e_ref,
                  wa_src_ref, wa_e_ref, ba1_ref, wa2_ref, ba2_ref,
                  wu_src_ref, wu_e_ref, bu1_ref, wu2_ref, bu2_ref,
                  gru_ue_w_ref, gru_ue_b_ref, gru_ap_w_ref, gru_ap_b_ref,
                  we_u_ref, we_a_ref, we_e_ref, be1_ref, we2_ref, be2_ref,
                  h_ue_out_ref, h_ap_out_ref, e_out_ref):
    f32 = jnp.float32
    hu = h_ue_ref[0]
    ha = h_ap_ref[0]
    e2 = e_ref[0]

    a_src = jnp.dot(ha, wa_src_ref[...], preferred_element_type=f32)
    t = jnp.dot(e2, wa_e_ref[...], preferred_element_type=f32)
    t = t.reshape(L, K, H) + a_src[:, None, :] + ba1_ref[...]
    t = jax.nn.relu(t).reshape(L * K, H)
    m = jnp.dot(t, wa2_ref[...], preferred_element_type=f32)
    m_ue = m.reshape(L, K, H).sum(axis=0) * (1.0 / L) + ba2_ref[...]

    u_src = jnp.dot(hu, wu_src_ref[...], preferred_element_type=f32)
    t = jnp.dot(e2, wu_e_ref[...], preferred_element_type=f32)
    t = t.reshape(L, K, H) + u_src[None, :, :] + bu1_ref[...]
    t = jax.nn.relu(t).reshape(L * K, H)
    m = jnp.dot(t, wu2_ref[...], preferred_element_type=f32)
    m_ap = m.reshape(L, K, H).sum(axis=1) * (1.0 / K) + bu2_ref[...]

    def gru(x, h, w_ref, b_ref):
        gi = jnp.dot(x, w_ref[:H, :], preferred_element_type=f32)
        gh = jnp.dot(h, w_ref[H:, :], preferred_element_type=f32)
        bi = b_ref[0:1, :]
        bh = b_ref[1:2, :]
        r = jax.nn.sigmoid(gi[:, :H] + gh[:, :H] + bi[:, :H] + bh[:, :H])
        z = jax.nn.sigmoid(gi[:, H:2 * H] + gh[:, H:2 * H] + bi[:, H:2 * H] + bh[:, H:2 * H])
        n = jnp.tanh(gi[:, 2 * H:] + bi[:, 2 * H:] + r * (gh[:, 2 * H:] + bh[:, 2 * H:]))
        return (1.0 - z) * n + z * h

    hu_new = gru(m_ue, hu, gru_ue_w_ref, gru_ue_b_ref)
    ha_new = gru(m_ap, ha, gru_ap_w_ref, gru_ap_b_ref)
    h_ue_out_ref[0] = hu_new
    h_ap_out_ref[0] = ha_new

    s_u = jnp.dot(hu_new, we_u_ref[...], preferred_element_type=f32)
    s_a = jnp.dot(ha_new, we_a_ref[...], preferred_element_type=f32)
    t = jnp.dot(e2, we_e_ref[...], preferred_element_type=f32)
    t = t.reshape(L, K, H) + s_u[None, :, :] + s_a[:, None, :] + be1_ref[...]
    t = jax.nn.relu(t).reshape(L * K, H)
    e_out_ref[0] = jnp.dot(t, we2_ref[...], preferred_element_type=f32) + be2_ref[...]


def kernel(h_ue, h_ap, e, W_a2u_1, b_a2u_1, W_a2u_2, b_a2u_2,
           W_u2a_1, b_u2a_1, W_u2a_2, b_u2a_2,
           Wih_ue, bih_ue, Whh_ue, bhh_ue, Wih_ap, bih_ap, Whh_ap, bhh_ap,
           W_e_1, b_e_1, W_e_2, b_e_2):
    wa_src = W_a2u_1[:, :H].T
    wa_e = W_a2u_1[:, H:].T
    wu_src = W_u2a_1[:, :H].T
    wu_e = W_u2a_1[:, H:].T
    we_u = W_e_1[:, :H].T
    we_a = W_e_1[:, H:2 * H].T
    we_e = W_e_1[:, 2 * H:].T
    gru_ue_w = jnp.concatenate([Wih_ue.T, Whh_ue.T], axis=0)
    gru_ue_b = jnp.stack([bih_ue, bhh_ue], axis=0)
    gru_ap_w = jnp.concatenate([Wih_ap.T, Whh_ap.T], axis=0)
    gru_ap_b = jnp.stack([bih_ap, bhh_ap], axis=0)

    ba1 = b_a2u_1.reshape(1, H)
    ba2 = b_a2u_2.reshape(1, H)
    bu1 = b_u2a_1.reshape(1, H)
    bu2 = b_u2a_2.reshape(1, H)
    be1 = b_e_1.reshape(1, H)
    be2 = b_e_2.reshape(1, H)

    def bspec(block, index_map):
        return pl.BlockSpec(block, index_map)

    batch3 = lambda s: pl.BlockSpec((1,) + s, lambda b: (b, 0, 0))
    fixed = lambda s: pl.BlockSpec(s, lambda b: (0,) * len(s))

    out_shapes = (
        jax.ShapeDtypeStruct((B, K, H), jnp.float32),
        jax.ShapeDtypeStruct((B, L, H), jnp.float32),
        jax.ShapeDtypeStruct((B, L * K, H), jnp.float32),
    )

    in_specs = [
            batch3((K, H)), batch3((L, H)), batch3((L * K, H)),
            fixed((H, H)), fixed((H, H)), fixed((1, H)), fixed((H, H)), fixed((1, H)),
            fixed((H, H)), fixed((H, H)), fixed((1, H)), fixed((H, H)), fixed((1, H)),
            fixed((2 * H, 3 * H)), fixed((2, 3 * H)),
            fixed((2 * H, 3 * H)), fixed((2, 3 * H)),
            fixed((H, H)), fixed((H, H)), fixed((H, H)), fixed((1, H)), fixed((H, H)),
            fixed((1, H)),
        ]

    return pl.pallas_call(
        _fused_kernel,
        grid=(B,),
        in_specs=in_specs,
        out_specs=[batch3((K, H)), batch3((L, H)), batch3((L * K, H))],
        out_shape=out_shapes,
        compiler_params=pltpu.CompilerParams(
            dimension_semantics=("parallel",),
        ),
    )(h_ue, h_ap, e,
      wa_src, wa_e, ba1, W_a2u_2.T, ba2,
      wu_src, wu_e, bu1, W_u2a_2.T, bu2,
      gru_ue_w, gru_ue_b, gru_ap_w, gru_ap_b,
      we_u, we_a, we_e, be1, W_e_2.T, be2)


# speedup vs baseline: 1.0046x; 1.0046x over previous
import jax
import jax.numpy as jnp
from jax import lax
from jax.experimental import pallas as pl
from jax.experimental.pallas import tpu as pltpu

B, K, L, H = 64, 64, 64, 64

_DNT = (((1,), (1,)), ((), ()))


def _mmT(x, w):
    return lax.dot_general(x, w, _DNT, preferred_element_type=jnp.float32)


def _fused_kernel(h_ue_ref, h_ap_ref, e_ref,
                  wa1_ref, ba1_ref, wa2_ref, ba2_ref,
                  wu1_ref, bu1_ref, wu2_ref, bu2_ref,
                  wih_ue_ref, bih_ue_ref, whh_ue_ref, bhh_ue_ref,
                  wih_ap_ref, bih_ap_ref, whh_ap_ref, bhh_ap_ref,
                  we1_ref, be1_ref, we2_ref, be2_ref,
                  h_ue_out_ref, h_ap_out_ref, e_out_ref):
    hu = h_ue_ref[0]
    ha = h_ap_ref[0]
    e2 = e_ref[0]

    a_src = _mmT(ha, wa1_ref[:, :H])
    t = _mmT(e2, wa1_ref[:, H:])
    t = t.reshape(L, K, H) + a_src[:, None, :] + ba1_ref[...]
    t = jax.nn.relu(t).reshape(L * K, H)
    m = _mmT(t, wa2_ref[...])
    m_ue = m.reshape(L, K, H).sum(axis=0) * (1.0 / L) + ba2_ref[...]

    u_src = _mmT(hu, wu1_ref[:, :H])
    t = _mmT(e2, wu1_ref[:, H:])
    t = t.reshape(L, K, H) + u_src[None, :, :] + bu1_ref[...]
    t = jax.nn.relu(t).reshape(L * K, H)
    m = _mmT(t, wu2_ref[...])
    m_ap = m.reshape(L, K, H).sum(axis=1) * (1.0 / K) + bu2_ref[...]

    def gru(x, h, wih_ref, bih_ref, whh_ref, bhh_ref):
        gi = _mmT(x, wih_ref[...]) + bih_ref[...]
        gh = _mmT(h, whh_ref[...]) + bhh_ref[...]
        r = jax.nn.sigmoid(gi[:, :H] + gh[:, :H])
        z = jax.nn.sigmoid(gi[:, H:2 * H] + gh[:, H:2 * H])
        n = jnp.tanh(gi[:, 2 * H:] + r * gh[:, 2 * H:])
        return (1.0 - z) * n + z * h

    hu_new = gru(m_ue, hu, wih_ue_ref, bih_ue_ref, whh_ue_ref, bhh_ue_ref)
    ha_new = gru(m_ap, ha, wih_ap_ref, bih_ap_ref, whh_ap_ref, bhh_ap_ref)
    h_ue_out_ref[0] = hu_new
    h_ap_out_ref[0] = ha_new

    s_u = _mmT(hu_new, we1_ref[:, :H])
    s_a = _mmT(ha_new, we1_ref[:, H:2 * H])
    t = _mmT(e2, we1_ref[:, 2 * H:])
    t = t.reshape(L, K, H) + s_u[None, :, :] + s_a[:, None, :] + be1_ref[...]
    t = jax.nn.relu(t).reshape(L * K, H)
    e_out_ref[0] = _mmT(t, we2_ref[...]) + be2_ref[...]


def kernel(h_ue, h_ap, e, W_a2u_1, b_a2u_1, W_a2u_2, b_a2u_2,
           W_u2a_1, b_u2a_1, W_u2a_2, b_u2a_2,
           Wih_ue, bih_ue, Whh_ue, bhh_ue, Wih_ap, bih_ap, Whh_ap, bhh_ap,
           W_e_1, b_e_1, W_e_2, b_e_2):
    batch3 = lambda s: pl.BlockSpec((1,) + s, lambda b: (b, 0, 0))
    fixed = lambda s: pl.BlockSpec(s, lambda b: (0,) * len(s))

    out_shapes = (
        jax.ShapeDtypeStruct((B, K, H), jnp.float32),
        jax.ShapeDtypeStruct((B, L, H), jnp.float32),
        jax.ShapeDtypeStruct((B, L * K, H), jnp.float32),
    )

    in_specs = [
        batch3((K, H)), batch3((L, H)), batch3((L * K, H)),
        fixed((H, 2 * H)), fixed((1, H)), fixed((H, H)), fixed((1, H)),
        fixed((H, 2 * H)), fixed((1, H)), fixed((H, H)), fixed((1, H)),
        fixed((3 * H, H)), fixed((1, 3 * H)), fixed((3 * H, H)), fixed((1, 3 * H)),
        fixed((3 * H, H)), fixed((1, 3 * H)), fixed((3 * H, H)), fixed((1, 3 * H)),
        fixed((H, 3 * H)), fixed((1, H)), fixed((H, H)), fixed((1, H)),
    ]

    return pl.pallas_call(
        _fused_kernel,
        grid=(B,),
        in_specs=in_specs,
        out_specs=[batch3((K, H)), batch3((L, H)), batch3((L * K, H))],
        out_shape=out_shapes,
        compiler_params=pltpu.CompilerParams(
            dimension_semantics=("arbitrary",),
        ),
    )(h_ue, h_ap, e,
      W_a2u_1, b_a2u_1.reshape(1, H), W_a2u_2, b_a2u_2.reshape(1, H),
      W_u2a_1, b_u2a_1.reshape(1, H), W_u2a_2, b_u2a_2.reshape(1, H),
      Wih_ue, bih_ue.reshape(1, 3 * H), Whh_ue, bhh_ue.reshape(1, 3 * H),
      Wih_ap, bih_ap.reshape(1, 3 * H), Whh_ap, bhh_ap.reshape(1, 3 * H),
      W_e_1, b_e_1.reshape(1, H), W_e_2, b_e_2.reshape(1, H))
